# baseline (speedup 1.0000x reference)
import functools

import jax
import jax.numpy as jnp
from jax import lax
from jax.experimental import pallas as pl
from jax.experimental.pallas import tpu as pltpu


def _cell(gates, c_prev, H):
    i_g = jax.nn.sigmoid(gates[:, 0:H])
    f_g = jax.nn.sigmoid(gates[:, H:2 * H])
    g_g = jnp.tanh(gates[:, 2 * H:3 * H])
    o_g = jax.nn.sigmoid(gates[:, 3 * H:4 * H])
    c_new = f_g * c_prev + i_g * g_g
    h_new = o_g * jnp.tanh(c_new)
    return h_new, c_new


def _mega_kernel(widx_ref, sidx_ref, emb_ref, len_ref,
                 wih_ref, wb_ref, whf_ref, whb_ref,
                 sih_ref, sb_ref, swhf_ref, swhb_ref,
                 wgf_ref, wgs_ref, bg_ref,
                 out_ref,
                 xw_ref, xs_ref, fwf_ref, fwb_ref, swf_ref, swb_ref,
                 table, gx, obuf, semt, semo,
                 *, T, B, S, L, D, H, VOCAB):
    BS = B * S
    NW = T * B
    NS = L * BS
    four_h = 4 * H
    P = T + 1

    NDMA = 4
    rows_per = -(-VOCAB // NDMA)
    rows_per += (-rows_per) % 8
    cps = []
    for d in range(NDMA):
        v0 = d * rows_per
        v1 = min(VOCAB, (d + 1) * rows_per)
        if v0 >= v1:
            continue
        cps.append(pltpu.make_async_copy(
            emb_ref.at[v0:v1, :], table.at[v0:v1, 0, :], semt.at[d]))
    for cp in cps:
        cp.start()
    for cp in cps:
        cp.wait()

    def gather_chunk(dst_ref, idx_ref, base, count):
        for j in range(0, count, 16):
            rows = jnp.concatenate(
                [jnp.stack([table[idx_ref[base + j + 8 * v + u], 0]
                            for u in range(8)], axis=0)
                 for v in range(2)], axis=0)
            dst_ref[pl.ds(pl.multiple_of(base + j, 16), 16)] = \
                rows.astype(jnp.bfloat16)

    CH = 256
    s_chunks = [(k * CH, min(CH, NS - k * CH))
                for k in range(-(-NS // CH))]
    w_chunks = [(k * CH, min(CH, NW - k * CH))
                for k in range(-(-NW // CH))]

    gather_chunk(xs_ref, sidx_ref, s_chunks[0][0], s_chunks[0][1])
    for k, (r0, sz) in enumerate(s_chunks):
        gx[r0:r0 + sz, :] = jnp.dot(
            xs_ref[r0:r0 + sz, :], sih_ref[...],
            preferred_element_type=jnp.float32) + sb_ref[...]
        if k + 1 < len(s_chunks):
            gather_chunk(xs_ref, sidx_ref,
                         s_chunks[k + 1][0], s_chunks[k + 1][1])

    whf = swhf_ref[...]
    whb = swhb_ref[...]
    zs = jnp.zeros((BS, H), jnp.float32)
    h_f, c_f, h_b, c_b = zs, zs, zs, zs
    if S * L < T:
        for t0 in range(S * L, T):
            z = jnp.zeros((B, H), jnp.float32)
            swf_ref[t0:t0 + P * (B - 1) + 1:P, :] = z
            swb_ref[t0:t0 + P * (B - 1) + 1:P, :] = z
    for l in range(L):
        lb = L - 1 - l
        gf = gx[l * BS:(l + 1) * BS, 0:four_h] + jnp.dot(
            h_f.astype(jnp.bfloat16), whf,
            preferred_element_type=jnp.float32)
        gb = gx[lb * BS:(lb + 1) * BS, four_h:2 * four_h] + jnp.dot(
            h_b.astype(jnp.bfloat16), whb,
            preferred_element_type=jnp.float32)
        h_f, c_f = _cell(gf, c_f, H)
        h_b, c_b = _cell(gb, c_b, H)
        for s in range(S):
            if s * L + l < T:
                t0 = s * L + l
                swf_ref[t0:t0 + P * (B - 1) + 1:P, :] = h_f[s * B:(s + 1) * B]
            if s * L + lb < T:
                t0 = s * L + lb
                swb_ref[t0:t0 + P * (B - 1) + 1:P, :] = h_b[s * B:(s + 1) * B]
        if l < len(w_chunks):
            gather_chunk(xw_ref, widx_ref, w_chunks[l][0], w_chunks[l][1])
    for i in range(L, len(w_chunks)):
        gather_chunk(xw_ref, widx_ref, w_chunks[i][0], w_chunks[i][1])

    gx[0:NW, :] = jnp.dot(
        xw_ref[...], wih_ref[...],
        preferred_element_type=jnp.float32) + wb_ref[...]
    whf = whf_ref[...]
    whb = whb_ref[...]
    lens = len_ref[...]
    zw = jnp.zeros((B, H), jnp.float32)
    h_f, c_f, h_b, c_b = zw, zw, zw, zw
    for t in range(T):
        tb = T - 1 - t
        gf = gx[t * B:(t + 1) * B, 0:four_h] + jnp.dot(
            h_f.astype(jnp.bfloat16), whf,
            preferred_element_type=jnp.float32)
        gb = gx[tb * B:(tb + 1) * B, four_h:2 * four_h] + jnp.dot(
            h_b.astype(jnp.bfloat16), whb,
            preferred_element_type=jnp.float32)
        hf_c, cf_c = _cell(gf, c_f, H)
        hb_c, cb_c = _cell(gb, c_b, H)
        vf = t < lens
        vb = tb < lens
        fwf_ref[t:t + P * (B - 1) + 1:P, :] = jnp.where(vf, hf_c, 0.0)
        fwb_ref[tb:tb + P * (B - 1) + 1:P, :] = jnp.where(vb, hb_c, 0.0)
        h_f = jnp.where(vf, hf_c, h_f)
        c_f = jnp.where(vf, cf_c, c_f)
        h_b = jnp.where(vb, hb_c, h_b)
        c_b = jnp.where(vb, cb_c, c_b)

    wgf = wgf_ref[...]
    wgs = wgs_ref[...]
    bg = bg_ref[...]
    BB = min(B, max(1, 512 // T))
    for b0 in range(0, B, BB):
        f = jnp.concatenate(
            [jnp.concatenate([fwf_ref[b * P:b * P + T, :],
                              fwb_ref[b * P:b * P + T, :]], axis=1)
             for b in range(b0, b0 + BB)], axis=0)
        s = jnp.concatenate(
            [jnp.concatenate([swf_ref[b * P:b * P + T, :],
                              swb_ref[b * P:b * P + T, :]], axis=1)
             for b in range(b0, b0 + BB)], axis=0)
        logits = (
            jnp.dot(f.astype(jnp.bfloat16), wgf,
                    preferred_element_type=jnp.float32)
            + jnp.dot(s.astype(jnp.bfloat16), wgs,
                      preferred_element_type=jnp.float32)
            + bg
        )
        g = jax.nn.sigmoid(logits)
        obuf[b0 * T:(b0 + BB) * T, :] = g * f + (1.0 - g) * s
        ocp = pltpu.make_async_copy(
            obuf.at[b0 * T:(b0 + BB) * T, :],
            out_ref.at[b0 * T:(b0 + BB) * T, :],
            semo.at[(b0 // BB) % 2])
        ocp.start()
        if b0 // BB >= 1:
            p0 = (b0 - BB) * T
            pltpu.make_async_copy(
                obuf.at[p0:p0 + BB * T, :],
                out_ref.at[p0:p0 + BB * T, :],
                semo.at[(b0 // BB - 1) % 2]).wait()
    last = ((B - 1) // BB) * BB
    pltpu.make_async_copy(
        obuf.at[last * T:B * T, :],
        out_ref.at[last * T:B * T, :],
        semo.at[(last // BB) % 2]).wait()


def kernel(word_inputs, sent_tokens, word_seq_lengths, seq_token_masks,
           embedding, lstm_w_ih, lstm_b, lstm_w_hh_f, lstm_w_hh_b,
           sent_lstm_w_ih, sent_lstm_b, sent_lstm_w_hh_f, sent_lstm_w_hh_b,
           w_gate_f, w_gate_s, b_gate):
    B, T = word_inputs.shape
    _, S, L = sent_tokens.shape
    VOCAB, D = embedding.shape
    H = lstm_w_hh_f.shape[0]
    two_h = 2 * H

    widx = jnp.transpose(word_inputs).reshape(-1).astype(jnp.int32)
    sidx = jnp.transpose(sent_tokens, (2, 1, 0)).reshape(-1).astype(jnp.int32)
    lens = word_seq_lengths.astype(jnp.int32).reshape(B, 1)

    out2d = pl.pallas_call(
        functools.partial(_mega_kernel, T=T, B=B, S=S, L=L, D=D, H=H,
                          VOCAB=VOCAB),
        out_shape=jax.ShapeDtypeStruct((T * B, two_h), jnp.float32),
        in_specs=[
            pl.BlockSpec(memory_space=pltpu.SMEM),
            pl.BlockSpec(memory_space=pltpu.SMEM),
            pl.BlockSpec(memory_space=pl.ANY),
            pl.BlockSpec(memory_space=pltpu.VMEM),
            pl.BlockSpec(memory_space=pltpu.VMEM),
            pl.BlockSpec(memory_space=pltpu.VMEM),
            pl.BlockSpec(memory_space=pltpu.VMEM),
            pl.BlockSpec(memory_space=pltpu.VMEM),
            pl.BlockSpec(memory_space=pltpu.VMEM),
            pl.BlockSpec(memory_space=pltpu.VMEM),
            pl.BlockSpec(memory_space=pltpu.VMEM),
            pl.BlockSpec(memory_space=pltpu.VMEM),
            pl.BlockSpec(memory_space=pltpu.VMEM),
            pl.BlockSpec(memory_space=pltpu.VMEM),
            pl.BlockSpec(memory_space=pltpu.VMEM),
        ],
        out_specs=pl.BlockSpec(memory_space=pl.ANY),
        scratch_shapes=[
            pltpu.VMEM((T * B, D), jnp.bfloat16),
            pltpu.VMEM((L * B * S, D), jnp.bfloat16),
            pltpu.VMEM(((T + 1) * B, H), jnp.float32),
            pltpu.VMEM(((T + 1) * B, H), jnp.float32),
            pltpu.VMEM(((T + 1) * B, H), jnp.float32),
            pltpu.VMEM(((T + 1) * B, H), jnp.float32),
            pltpu.VMEM((VOCAB, 1, D), jnp.float32),
            pltpu.VMEM((max(T * B, L * B * S), 8 * H), jnp.float32),
            pltpu.VMEM((T * B, 2 * H), jnp.float32),
            pltpu.SemaphoreType.DMA((4,)),
            pltpu.SemaphoreType.DMA((2,)),
        ],
        compiler_params=pltpu.CompilerParams(
            vmem_limit_bytes=56 * 1024 * 1024),
    )(widx, sidx, embedding, lens, lstm_w_ih, lstm_b, lstm_w_hh_f,
      lstm_w_hh_b, sent_lstm_w_ih, sent_lstm_b, sent_lstm_w_hh_f,
      sent_lstm_w_hh_b, w_gate_f, w_gate_s, b_gate)

    return out2d.reshape(B, T, two_h)


# speedup vs baseline: 1.0422x; 1.0422x over previous
import functools

import jax
import jax.numpy as jnp
from jax import lax
from jax.experimental import pallas as pl
from jax.experimental.pallas import tpu as pltpu


def _cell(gates, c_prev, H):
    i_g = jax.nn.sigmoid(gates[:, 0:H])
    f_g = jax.nn.sigmoid(gates[:, H:2 * H])
    g_g = jnp.tanh(gates[:, 2 * H:3 * H])
    o_g = jax.nn.sigmoid(gates[:, 3 * H:4 * H])
    c_new = f_g * c_prev + i_g * g_g
    h_new = o_g * jnp.tanh(c_new)
    return h_new, c_new


def _mega_kernel(widx_ref, sidx_ref, emb_ref, len_ref,
                 wih_ref, wb_ref, whf_ref, whb_ref,
                 sih_ref, sb_ref, swhf_ref, swhb_ref,
                 wgf_ref, wgs_ref, bg_ref,
                 out_ref,
                 xw_ref, xs_ref, fwf_ref, fwb_ref, swf_ref, swb_ref,
                 table, gx, semt,
                 *, T, B, S, L, D, H, VOCAB):
    BS = B * S
    NW = T * B
    NS = L * BS
    four_h = 4 * H
    P = T + 1

    NDMA = 4
    rows_per = -(-VOCAB // NDMA)
    rows_per += (-rows_per) % 8
    cps = []
    for d in range(NDMA):
        v0 = d * rows_per
        v1 = min(VOCAB, (d + 1) * rows_per)
        if v0 >= v1:
            continue
        cps.append(pltpu.make_async_copy(
            emb_ref.at[v0:v1, :], table.at[v0:v1, 0, :], semt.at[d]))
    for cp in cps:
        cp.start()
    for cp in cps:
        cp.wait()

    def gather_chunk(dst_ref, idx_ref, base, count):
        for j in range(0, count, 16):
            rows = jnp.concatenate(
                [jnp.stack([table[idx_ref[base + j + 8 * v + u], 0]
                            for u in range(8)], axis=0)
                 for v in range(2)], axis=0)
            dst_ref[pl.ds(pl.multiple_of(base + j, 16), 16)] = \
                rows.astype(jnp.bfloat16)

    CH = 512
    s_chunks = [(k * CH, min(CH, NS - k * CH))
                for k in range(-(-NS // CH))]
    w_chunks = [(k * CH, min(CH, NW - k * CH))
                for k in range(-(-NW // CH))]

    gather_chunk(xs_ref, sidx_ref, s_chunks[0][0], s_chunks[0][1])
    for k, (r0, sz) in enumerate(s_chunks):
        gx[r0:r0 + sz, :] = jnp.dot(
            xs_ref[r0:r0 + sz, :], sih_ref[...],
            preferred_element_type=jnp.float32) + sb_ref[...]
        if k + 1 < len(s_chunks):
            gather_chunk(xs_ref, sidx_ref,
                         s_chunks[k + 1][0], s_chunks[k + 1][1])

    whf = swhf_ref[...]
    whb = swhb_ref[...]
    zs = jnp.zeros((BS, H), jnp.float32)
    h_f, c_f, h_b, c_b = zs, zs, zs, zs
    if S * L < T:
        for t0 in range(S * L, T):
            z = jnp.zeros((B, H), jnp.float32)
            swf_ref[t0:t0 + P * (B - 1) + 1:P, :] = z
            swb_ref[t0:t0 + P * (B - 1) + 1:P, :] = z
    for l in range(L):
        lb = L - 1 - l
        gf = gx[l * BS:(l + 1) * BS, 0:four_h] + jnp.dot(
            h_f.astype(jnp.bfloat16), whf,
            preferred_element_type=jnp.float32)
        gb = gx[lb * BS:(lb + 1) * BS, four_h:2 * four_h] + jnp.dot(
            h_b.astype(jnp.bfloat16), whb,
            preferred_element_type=jnp.float32)
        h_f, c_f = _cell(gf, c_f, H)
        h_b, c_b = _cell(gb, c_b, H)
        for s in range(S):
            if s * L + l < T:
                t0 = s * L + l
                swf_ref[t0:t0 + P * (B - 1) + 1:P, :] = h_f[s * B:(s + 1) * B]
            if s * L + lb < T:
                t0 = s * L + lb
                swb_ref[t0:t0 + P * (B - 1) + 1:P, :] = h_b[s * B:(s + 1) * B]
        if l < len(w_chunks):
            gather_chunk(xw_ref, widx_ref, w_chunks[l][0], w_chunks[l][1])
    for i in range(L, len(w_chunks)):
        gather_chunk(xw_ref, widx_ref, w_chunks[i][0], w_chunks[i][1])

    gx[0:NW, :] = jnp.dot(
        xw_ref[...], wih_ref[...],
        preferred_element_type=jnp.float32) + wb_ref[...]
    whf = whf_ref[...]
    whb = whb_ref[...]
    lens = len_ref[...]
    zw = jnp.zeros((B, H), jnp.float32)
    h_f, c_f, h_b, c_b = zw, zw, zw, zw
    for t in range(T):
        tb = T - 1 - t
        gf = gx[t * B:(t + 1) * B, 0:four_h] + jnp.dot(
            h_f.astype(jnp.bfloat16), whf,
            preferred_element_type=jnp.float32)
        gb = gx[tb * B:(tb + 1) * B, four_h:2 * four_h] + jnp.dot(
            h_b.astype(jnp.bfloat16), whb,
            preferred_element_type=jnp.float32)
        hf_c, cf_c = _cell(gf, c_f, H)
        hb_c, cb_c = _cell(gb, c_b, H)
        vf = t < lens
        vb = tb < lens
        fwf_ref[t:t + P * (B - 1) + 1:P, :] = jnp.where(vf, hf_c, 0.0)
        fwb_ref[tb:tb + P * (B - 1) + 1:P, :] = jnp.where(vb, hb_c, 0.0)
        h_f = jnp.where(vf, hf_c, h_f)
        c_f = jnp.where(vf, cf_c, c_f)
        h_b = jnp.where(vb, hb_c, h_b)
        c_b = jnp.where(vb, cb_c, c_b)

    wgf = wgf_ref[...]
    wgs = wgs_ref[...]
    bg = bg_ref[...]
    BB = min(B, max(1, 512 // T))
    for b0 in range(0, B, BB):
        f = jnp.concatenate(
            [jnp.concatenate([fwf_ref[b * P:b * P + T, :],
                              fwb_ref[b * P:b * P + T, :]], axis=1)
             for b in range(b0, b0 + BB)], axis=0)
        s = jnp.concatenate(
            [jnp.concatenate([swf_ref[b * P:b * P + T, :],
                              swb_ref[b * P:b * P + T, :]], axis=1)
             for b in range(b0, b0 + BB)], axis=0)
        logits = (
            jnp.dot(f.astype(jnp.bfloat16), wgf,
                    preferred_element_type=jnp.float32)
            + jnp.dot(s.astype(jnp.bfloat16), wgs,
                      preferred_element_type=jnp.float32)
            + bg
        )
        g = jax.nn.sigmoid(logits)
        out_ref[b0 * T:(b0 + BB) * T, :] = g * f + (1.0 - g) * s


def kernel(word_inputs, sent_tokens, word_seq_lengths, seq_token_masks,
           embedding, lstm_w_ih, lstm_b, lstm_w_hh_f, lstm_w_hh_b,
           sent_lstm_w_ih, sent_lstm_b, sent_lstm_w_hh_f, sent_lstm_w_hh_b,
           w_gate_f, w_gate_s, b_gate):
    B, T = word_inputs.shape
    _, S, L = sent_tokens.shape
    VOCAB, D = embedding.shape
    H = lstm_w_hh_f.shape[0]
    two_h = 2 * H

    widx = jnp.transpose(word_inputs).reshape(-1).astype(jnp.int32)
    sidx = jnp.transpose(sent_tokens, (2, 1, 0)).reshape(-1).astype(jnp.int32)
    lens = word_seq_lengths.astype(jnp.int32).reshape(B, 1)

    out2d = pl.pallas_call(
        functools.partial(_mega_kernel, T=T, B=B, S=S, L=L, D=D, H=H,
                          VOCAB=VOCAB),
        out_shape=jax.ShapeDtypeStruct((T * B, two_h), jnp.float32),
        in_specs=[
            pl.BlockSpec(memory_space=pltpu.SMEM),
            pl.BlockSpec(memory_space=pltpu.SMEM),
            pl.BlockSpec(memory_space=pl.ANY),
            pl.BlockSpec(memory_space=pltpu.VMEM),
            pl.BlockSpec(memory_space=pltpu.VMEM),
            pl.BlockSpec(memory_space=pltpu.VMEM),
            pl.BlockSpec(memory_space=pltpu.VMEM),
            pl.BlockSpec(memory_space=pltpu.VMEM),
            pl.BlockSpec(memory_space=pltpu.VMEM),
            pl.BlockSpec(memory_space=pltpu.VMEM),
            pl.BlockSpec(memory_space=pltpu.VMEM),
            pl.BlockSpec(memory_space=pltpu.VMEM),
            pl.BlockSpec(memory_space=pltpu.VMEM),
            pl.BlockSpec(memory_space=pltpu.VMEM),
            pl.BlockSpec(memory_space=pltpu.VMEM),
        ],
        out_specs=pl.BlockSpec(memory_space=pltpu.VMEM),
        scratch_shapes=[
            pltpu.VMEM((T * B, D), jnp.bfloat16),
            pltpu.VMEM((L * B * S, D), jnp.bfloat16),
            pltpu.VMEM(((T + 1) * B, H), jnp.float32),
            pltpu.VMEM(((T + 1) * B, H), jnp.float32),
            pltpu.VMEM(((T + 1) * B, H), jnp.float32),
            pltpu.VMEM(((T + 1) * B, H), jnp.float32),
            pltpu.VMEM((VOCAB, 1, D), jnp.float32),
            pltpu.VMEM((max(T * B, L * B * S), 8 * H), jnp.float32),
            pltpu.SemaphoreType.DMA((4,)),
        ],
        compiler_params=pltpu.CompilerParams(
            vmem_limit_bytes=56 * 1024 * 1024),
    )(widx, sidx, embedding, lens, lstm_w_ih, lstm_b, lstm_w_hh_f,
      lstm_w_hh_b, sent_lstm_w_ih, sent_lstm_b, sent_lstm_w_hh_f,
      sent_lstm_w_hh_b, w_gate_f, w_gate_s, b_gate)

    return out2d.reshape(B, T, two_h)
